# baseline (speedup 1.0000x reference)
import functools

import jax
import jax.numpy as jnp
from jax.experimental import pallas as pl
from jax.experimental.pallas import tpu as pltpu

N = 10000
F = 128
BM = 200
BM2 = 1000
EPS = 1e-5


def _mm_kernel(adj_ref, seq_ref, w1t_ref, h_ref, stats_ref):
    p = jnp.dot(adj_ref[...].astype(jnp.bfloat16), seq_ref[...],
                preferred_element_type=jnp.float32)
    h = jnp.dot(p, w1t_ref[...], preferred_element_type=jnp.float32)
    h_ref[...] = h

    @pl.when(pl.program_id(0) == 0)
    def _zero_stats():
        stats_ref[...] = jnp.zeros_like(stats_ref)

    stats_ref[0:1, :] += jnp.sum(h, axis=0, keepdims=True)
    stats_ref[1:2, :] += jnp.sum(h * h, axis=0, keepdims=True)


def _bn_kernel(h_ref, self_ref, w2at_ref, w2bt_ref, stats_ref, gb_ref,
               out_ref):
    inv_n = 1.0 / N
    mean = stats_ref[0:1, :] * inv_n
    var = stats_ref[1:2, :] * inv_n - mean * mean
    scale = gb_ref[0:1, :] * jax.lax.rsqrt(var + EPS)
    shift = gb_ref[1:2, :] - mean * scale
    t = jnp.tanh(h_ref[...] * scale + shift)
    out = jnp.dot(self_ref[...], w2at_ref[...],
                  preferred_element_type=jnp.float32)
    out += jnp.dot(t, w2bt_ref[...], preferred_element_type=jnp.float32)
    out_ref[...] = jnp.tanh(out)


@functools.partial(jax.jit, static_argnames=())
def kernel(seq_self, seq, adj, W1, W2, gamma, beta):
    w1t = W1.T
    w2at = W2[:, :F].T
    w2bt = W2[:, F:].T
    gb = jnp.zeros((8, F), jnp.float32).at[0].set(gamma).at[1].set(beta)

    h, stats = pl.pallas_call(
        _mm_kernel,
        grid=(N // BM,),
        in_specs=[
            pl.BlockSpec((BM, N), lambda i: (i, 0)),
            pl.BlockSpec((N, F), lambda i: (0, 0)),
            pl.BlockSpec((F, F), lambda i: (0, 0)),
        ],
        out_specs=[
            pl.BlockSpec((BM, F), lambda i: (i, 0)),
            pl.BlockSpec((8, F), lambda i: (0, 0)),
        ],
        out_shape=[
            jax.ShapeDtypeStruct((N, F), jnp.float32),
            jax.ShapeDtypeStruct((8, F), jnp.float32),
        ],
        compiler_params=pltpu.CompilerParams(
            dimension_semantics=("arbitrary",)),
    )(adj, seq.astype(jnp.bfloat16), w1t)

    out = pl.pallas_call(
        _bn_kernel,
        grid=(N // BM2,),
        in_specs=[
            pl.BlockSpec((BM2, F), lambda i: (i, 0)),
            pl.BlockSpec((BM2, F), lambda i: (i, 0)),
            pl.BlockSpec((F, F), lambda i: (0, 0)),
            pl.BlockSpec((F, F), lambda i: (0, 0)),
            pl.BlockSpec((8, F), lambda i: (0, 0)),
            pl.BlockSpec((8, F), lambda i: (0, 0)),
        ],
        out_specs=pl.BlockSpec((BM2, F), lambda i: (i, 0)),
        out_shape=jax.ShapeDtypeStruct((N, F), jnp.float32),
        compiler_params=pltpu.CompilerParams(
            dimension_semantics=("arbitrary",)),
    )(h, seq_self, w2at, w2bt, stats, gb)
    return out


# speedup vs baseline: 1.0243x; 1.0243x over previous
import functools

import jax
import jax.numpy as jnp
from jax.experimental import pallas as pl
from jax.experimental.pallas import tpu as pltpu

N = 10000
F = 128
BM = 200
BM2 = 1000
EPS = 1e-5


def _mm_kernel(adj_ref, seq_ref, w1t_ref, h_ref, stats_ref):
    p = jnp.dot(adj_ref[...], seq_ref[...],
                preferred_element_type=jnp.float32)
    h = jnp.dot(p, w1t_ref[...], preferred_element_type=jnp.float32)
    h_ref[...] = h

    @pl.when(pl.program_id(0) == 0)
    def _zero_stats():
        stats_ref[...] = jnp.zeros_like(stats_ref)

    stats_ref[0:1, :] += jnp.sum(h, axis=0, keepdims=True)
    stats_ref[1:2, :] += jnp.sum(h * h, axis=0, keepdims=True)


def _bn_kernel(h_ref, self_ref, w2at_ref, w2bt_ref, stats_ref, gb_ref,
               out_ref):
    inv_n = 1.0 / N
    mean = stats_ref[0:1, :] * inv_n
    var = stats_ref[1:2, :] * inv_n - mean * mean
    scale = gb_ref[0:1, :] * jax.lax.rsqrt(var + EPS)
    shift = gb_ref[1:2, :] - mean * scale
    t = jnp.tanh(h_ref[...] * scale + shift)
    out = jnp.dot(self_ref[...], w2at_ref[...],
                  preferred_element_type=jnp.float32)
    out += jnp.dot(t, w2bt_ref[...], preferred_element_type=jnp.float32)
    out_ref[...] = jnp.tanh(out)


@functools.partial(jax.jit, static_argnames=())
def kernel(seq_self, seq, adj, W1, W2, gamma, beta):
    w1t = W1.T
    w2at = W2[:, :F].T
    w2bt = W2[:, F:].T
    gb = jnp.zeros((8, F), jnp.float32).at[0].set(gamma).at[1].set(beta)

    h, stats = pl.pallas_call(
        _mm_kernel,
        grid=(N // BM,),
        in_specs=[
            pl.BlockSpec((BM, N), lambda i: (i, 0)),
            pl.BlockSpec((N, F), lambda i: (0, 0)),
            pl.BlockSpec((F, F), lambda i: (0, 0)),
        ],
        out_specs=[
            pl.BlockSpec((BM, F), lambda i: (i, 0)),
            pl.BlockSpec((8, F), lambda i: (0, 0)),
        ],
        out_shape=[
            jax.ShapeDtypeStruct((N, F), jnp.float32),
            jax.ShapeDtypeStruct((8, F), jnp.float32),
        ],
        compiler_params=pltpu.CompilerParams(
            dimension_semantics=("arbitrary",)),
    )(adj, seq, w1t)

    out = pl.pallas_call(
        _bn_kernel,
        grid=(N // BM2,),
        in_specs=[
            pl.BlockSpec((BM2, F), lambda i: (i, 0)),
            pl.BlockSpec((BM2, F), lambda i: (i, 0)),
            pl.BlockSpec((F, F), lambda i: (0, 0)),
            pl.BlockSpec((F, F), lambda i: (0, 0)),
            pl.BlockSpec((8, F), lambda i: (0, 0)),
            pl.BlockSpec((8, F), lambda i: (0, 0)),
        ],
        out_specs=pl.BlockSpec((BM2, F), lambda i: (i, 0)),
        out_shape=jax.ShapeDtypeStruct((N, F), jnp.float32),
        compiler_params=pltpu.CompilerParams(
            dimension_semantics=("arbitrary",)),
    )(h, seq_self, w2at, w2bt, stats, gb)
    return out


# speedup vs baseline: 1.0261x; 1.0017x over previous
import functools

import jax
import jax.numpy as jnp
from jax.experimental import pallas as pl
from jax.experimental.pallas import tpu as pltpu

N = 10000
F = 128
BM = 200
BM2 = 1000
EPS = 1e-5


def _mm_kernel(adj_ref, seq_ref, w1t_ref, h_ref, pstats_ref):
    p = jnp.dot(adj_ref[...], seq_ref[...],
                preferred_element_type=jnp.float32)
    h = jnp.dot(p, w1t_ref[...], preferred_element_type=jnp.float32)
    h_ref[...] = h
    pstats_ref[0, 0:1, :] = jnp.sum(h, axis=0, keepdims=True)
    pstats_ref[0, 1:2, :] = jnp.sum(h * h, axis=0, keepdims=True)


def _bn_kernel(h_ref, self_ref, w2at_ref, w2bt_ref, pstats_ref, gb_ref,
               out_ref):
    inv_n = 1.0 / N
    mean = jnp.sum(pstats_ref[:, 0, :], axis=0, keepdims=True) * inv_n
    var = jnp.sum(pstats_ref[:, 1, :], axis=0, keepdims=True) * inv_n \
        - mean * mean
    scale = gb_ref[0:1, :] * jax.lax.rsqrt(var + EPS)
    shift = gb_ref[1:2, :] - mean * scale
    t = jnp.tanh(h_ref[...] * scale + shift)
    out = jnp.dot(self_ref[...], w2at_ref[...],
                  preferred_element_type=jnp.float32)
    out += jnp.dot(t, w2bt_ref[...], preferred_element_type=jnp.float32)
    out_ref[...] = jnp.tanh(out)


@functools.partial(jax.jit, static_argnames=())
def kernel(seq_self, seq, adj, W1, W2, gamma, beta):
    w1t = W1.T
    w2at = W2[:, :F].T
    w2bt = W2[:, F:].T
    gb = jnp.zeros((8, F), jnp.float32).at[0].set(gamma).at[1].set(beta)

    h, stats = pl.pallas_call(
        _mm_kernel,
        grid=(N // BM,),
        in_specs=[
            pl.BlockSpec((BM, N), lambda i: (i, 0)),
            pl.BlockSpec((N, F), lambda i: (0, 0)),
            pl.BlockSpec((F, F), lambda i: (0, 0)),
        ],
        out_specs=[
            pl.BlockSpec((BM, F), lambda i: (i, 0)),
            pl.BlockSpec((1, 8, F), lambda i: (i, 0, 0)),
        ],
        out_shape=[
            jax.ShapeDtypeStruct((N, F), jnp.float32),
            jax.ShapeDtypeStruct((N // BM, 8, F), jnp.float32),
        ],
        compiler_params=pltpu.CompilerParams(
            dimension_semantics=("parallel",)),
    )(adj, seq, w1t)

    out = pl.pallas_call(
        _bn_kernel,
        grid=(N // BM2,),
        in_specs=[
            pl.BlockSpec((BM2, F), lambda i: (i, 0)),
            pl.BlockSpec((BM2, F), lambda i: (i, 0)),
            pl.BlockSpec((F, F), lambda i: (0, 0)),
            pl.BlockSpec((F, F), lambda i: (0, 0)),
            pl.BlockSpec((N // BM, 8, F), lambda i: (0, 0, 0)),
            pl.BlockSpec((8, F), lambda i: (0, 0)),
        ],
        out_specs=pl.BlockSpec((BM2, F), lambda i: (i, 0)),
        out_shape=jax.ShapeDtypeStruct((N, F), jnp.float32),
        compiler_params=pltpu.CompilerParams(
            dimension_semantics=("parallel",)),
    )(h, seq_self, w2at, w2bt, stats, gb)
    return out


# speedup vs baseline: 1.0490x; 1.0223x over previous
import functools

import jax
import jax.numpy as jnp
from jax.experimental import pallas as pl
from jax.experimental.pallas import tpu as pltpu

N = 10000
F = 128
BM = 400
BM2 = 1000
EPS = 1e-5


def _mm_kernel(adj_ref, seq_ref, w1t_ref, h_ref, pstats_ref):
    p = jnp.dot(adj_ref[...], seq_ref[...],
                preferred_element_type=jnp.float32)
    h = jnp.dot(p, w1t_ref[...], preferred_element_type=jnp.float32)
    h_ref[...] = h
    pstats_ref[0, 0:1, :] = jnp.sum(h, axis=0, keepdims=True)
    pstats_ref[0, 1:2, :] = jnp.sum(h * h, axis=0, keepdims=True)


def _bn_kernel(h_ref, self_ref, w2at_ref, w2bt_ref, pstats_ref, gb_ref,
               out_ref):
    inv_n = 1.0 / N
    mean = jnp.sum(pstats_ref[:, 0, :], axis=0, keepdims=True) * inv_n
    var = jnp.sum(pstats_ref[:, 1, :], axis=0, keepdims=True) * inv_n \
        - mean * mean
    scale = gb_ref[0:1, :] * jax.lax.rsqrt(var + EPS)
    shift = gb_ref[1:2, :] - mean * scale
    t = jnp.tanh(h_ref[...] * scale + shift)
    out = jnp.dot(self_ref[...], w2at_ref[...],
                  preferred_element_type=jnp.float32)
    out += jnp.dot(t, w2bt_ref[...], preferred_element_type=jnp.float32)
    out_ref[...] = jnp.tanh(out)


@functools.partial(jax.jit, static_argnames=())
def kernel(seq_self, seq, adj, W1, W2, gamma, beta):
    w1t = W1.T
    w2at = W2[:, :F].T
    w2bt = W2[:, F:].T
    gb = jnp.zeros((8, F), jnp.float32).at[0].set(gamma).at[1].set(beta)

    h, stats = pl.pallas_call(
        _mm_kernel,
        grid=(N // BM,),
        in_specs=[
            pl.BlockSpec((BM, N), lambda i: (i, 0)),
            pl.BlockSpec((N, F), lambda i: (0, 0)),
            pl.BlockSpec((F, F), lambda i: (0, 0)),
        ],
        out_specs=[
            pl.BlockSpec((BM, F), lambda i: (i, 0)),
            pl.BlockSpec((1, 8, F), lambda i: (i, 0, 0)),
        ],
        out_shape=[
            jax.ShapeDtypeStruct((N, F), jnp.float32),
            jax.ShapeDtypeStruct((N // BM, 8, F), jnp.float32),
        ],
        compiler_params=pltpu.CompilerParams(
            dimension_semantics=("parallel",)),
    )(adj, seq, w1t)

    out = pl.pallas_call(
        _bn_kernel,
        grid=(N // BM2,),
        in_specs=[
            pl.BlockSpec((BM2, F), lambda i: (i, 0)),
            pl.BlockSpec((BM2, F), lambda i: (i, 0)),
            pl.BlockSpec((F, F), lambda i: (0, 0)),
            pl.BlockSpec((F, F), lambda i: (0, 0)),
            pl.BlockSpec((N // BM, 8, F), lambda i: (0, 0, 0)),
            pl.BlockSpec((8, F), lambda i: (0, 0)),
        ],
        out_specs=pl.BlockSpec((BM2, F), lambda i: (i, 0)),
        out_shape=jax.ShapeDtypeStruct((N, F), jnp.float32),
        compiler_params=pltpu.CompilerParams(
            dimension_semantics=("parallel",)),
    )(h, seq_self, w2at, w2bt, stats, gb)
    return out


# speedup vs baseline: 1.0770x; 1.0267x over previous
import jax
import jax.numpy as jnp
from jax.experimental import pallas as pl
from jax.experimental.pallas import tpu as pltpu

N = 10000
F = 128
BM = 400
BM2 = 1000
NP1 = N // BM
NP2 = N // BM2
EPS = 1e-5


def _fused_kernel(adj_ref, seq_ref, w1t_ref, self_ref, w2at_ref, w2bt_ref,
                  gb_ref, out_ref, h_ref, stats_ref):
    i = pl.program_id(0)

    @pl.when(i < NP1)
    def _phase1():
        p = jnp.dot(adj_ref[...], seq_ref[...],
                    preferred_element_type=jnp.float32)
        h = jnp.dot(p, w1t_ref[...], preferred_element_type=jnp.float32)
        h_ref[pl.ds(i * BM, BM), :] = h

        @pl.when(i == 0)
        def _zero_stats():
            stats_ref[...] = jnp.zeros_like(stats_ref)

        stats_ref[0:1, :] += jnp.sum(h, axis=0, keepdims=True)
        stats_ref[1:2, :] += jnp.sum(h * h, axis=0, keepdims=True)

    @pl.when(i >= NP1)
    def _phase2():
        j = i - NP1
        inv_n = 1.0 / N
        mean = stats_ref[0:1, :] * inv_n
        var = stats_ref[1:2, :] * inv_n - mean * mean
        scale = gb_ref[0:1, :] * jax.lax.rsqrt(var + EPS)
        shift = gb_ref[1:2, :] - mean * scale
        hb = h_ref[pl.ds(j * BM2, BM2), :]
        t = jnp.tanh(hb * scale + shift)
        out = jnp.dot(self_ref[...], w2at_ref[...],
                      preferred_element_type=jnp.float32)
        out += jnp.dot(t, w2bt_ref[...], preferred_element_type=jnp.float32)
        out_ref[...] = jnp.tanh(out)


def kernel(seq_self, seq, adj, W1, W2, gamma, beta):
    w1t = W1.T
    w2at = W2[:, :F].T
    w2bt = W2[:, F:].T
    gb = jnp.zeros((8, F), jnp.float32).at[0].set(gamma).at[1].set(beta)

    out = pl.pallas_call(
        _fused_kernel,
        grid=(NP1 + NP2,),
        in_specs=[
            pl.BlockSpec((BM, N), lambda i: (jnp.minimum(i, NP1 - 1), 0)),
            pl.BlockSpec((N, F), lambda i: (0, 0)),
            pl.BlockSpec((F, F), lambda i: (0, 0)),
            pl.BlockSpec((BM2, F), lambda i: (jnp.maximum(i - NP1, 0), 0)),
            pl.BlockSpec((F, F), lambda i: (0, 0)),
            pl.BlockSpec((F, F), lambda i: (0, 0)),
            pl.BlockSpec((8, F), lambda i: (0, 0)),
        ],
        out_specs=pl.BlockSpec((BM2, F), lambda i: (jnp.maximum(i - NP1, 0), 0)),
        out_shape=jax.ShapeDtypeStruct((N, F), jnp.float32),
        scratch_shapes=[
            pltpu.VMEM((N, F), jnp.float32),
            pltpu.VMEM((8, F), jnp.float32),
        ],
        compiler_params=pltpu.CompilerParams(
            dimension_semantics=("arbitrary",)),
    )(adj, seq, w1t, seq_self, w2at, w2bt, gb)
    return out


# speedup vs baseline: 1.1044x; 1.0254x over previous
import jax
import jax.numpy as jnp
from jax.experimental import pallas as pl
from jax.experimental.pallas import tpu as pltpu

N = 10000
F = 128
BM = 400
BM2 = 2000
NP1 = N // BM
NP2 = N // BM2
EPS = 1e-5


def _fused_kernel(adj_ref, seq_ref, w1t_ref, self_ref, w2at_ref, w2bt_ref,
                  gb_ref, out_ref, h_ref, stats_ref):
    i = pl.program_id(0)

    @pl.when(i < NP1)
    def _phase1():
        p = jnp.dot(adj_ref[...], seq_ref[...],
                    preferred_element_type=jnp.float32)
        h = jnp.dot(p, w1t_ref[...], preferred_element_type=jnp.float32)
        h_ref[pl.ds(i * BM, BM), :] = h

        @pl.when(i == 0)
        def _zero_stats():
            stats_ref[...] = jnp.zeros_like(stats_ref)

        stats_ref[0:1, :] += jnp.sum(h, axis=0, keepdims=True)
        stats_ref[1:2, :] += jnp.sum(h * h, axis=0, keepdims=True)

    @pl.when(i >= NP1)
    def _phase2():
        j = i - NP1
        inv_n = 1.0 / N
        mean = stats_ref[0:1, :] * inv_n
        var = stats_ref[1:2, :] * inv_n - mean * mean
        scale = gb_ref[0:1, :] * jax.lax.rsqrt(var + EPS)
        shift = gb_ref[1:2, :] - mean * scale
        hb = h_ref[pl.ds(j * BM2, BM2), :]
        t = jnp.tanh(hb * scale + shift)
        out = jnp.dot(self_ref[...], w2at_ref[...],
                      preferred_element_type=jnp.float32)
        out += jnp.dot(t, w2bt_ref[...], preferred_element_type=jnp.float32)
        out_ref[...] = jnp.tanh(out)


def kernel(seq_self, seq, adj, W1, W2, gamma, beta):
    w1t = W1.T
    w2at = W2[:, :F].T
    w2bt = W2[:, F:].T
    gb = jnp.zeros((8, F), jnp.float32).at[0].set(gamma).at[1].set(beta)

    out = pl.pallas_call(
        _fused_kernel,
        grid=(NP1 + NP2,),
        in_specs=[
            pl.BlockSpec((BM, N), lambda i: (jnp.minimum(i, NP1 - 1), 0)),
            pl.BlockSpec((N, F), lambda i: (0, 0)),
            pl.BlockSpec((F, F), lambda i: (0, 0)),
            pl.BlockSpec((BM2, F), lambda i: (jnp.maximum(i - NP1, 0), 0)),
            pl.BlockSpec((F, F), lambda i: (0, 0)),
            pl.BlockSpec((F, F), lambda i: (0, 0)),
            pl.BlockSpec((8, F), lambda i: (0, 0)),
        ],
        out_specs=pl.BlockSpec((BM2, F), lambda i: (jnp.maximum(i - NP1, 0), 0)),
        out_shape=jax.ShapeDtypeStruct((N, F), jnp.float32),
        scratch_shapes=[
            pltpu.VMEM((N, F), jnp.float32),
            pltpu.VMEM((8, F), jnp.float32),
        ],
        compiler_params=pltpu.CompilerParams(
            dimension_semantics=("arbitrary",)),
    )(adj, seq, w1t, seq_self, w2at, w2bt, gb)
    return out


# speedup vs baseline: 1.1087x; 1.0039x over previous
import jax
import jax.numpy as jnp
from jax.experimental import pallas as pl
from jax.experimental.pallas import tpu as pltpu

N = 10000
F = 128
BM = 400
BM2 = 5000
NP1 = N // BM
NP2 = N // BM2
EPS = 1e-5


def _fused_kernel(adj_ref, seq_ref, w1t_ref, self_ref, w2at_ref, w2bt_ref,
                  gb_ref, out_ref, h_ref, stats_ref):
    i = pl.program_id(0)

    @pl.when(i < NP1)
    def _phase1():
        p = jnp.dot(adj_ref[...], seq_ref[...],
                    preferred_element_type=jnp.float32)
        h = jnp.dot(p, w1t_ref[...], preferred_element_type=jnp.float32)
        h_ref[pl.ds(i * BM, BM), :] = h

        @pl.when(i == 0)
        def _zero_stats():
            stats_ref[...] = jnp.zeros_like(stats_ref)

        stats_ref[0:1, :] += jnp.sum(h, axis=0, keepdims=True)
        stats_ref[1:2, :] += jnp.sum(h * h, axis=0, keepdims=True)

    @pl.when(i >= NP1)
    def _phase2():
        j = i - NP1
        inv_n = 1.0 / N
        mean = stats_ref[0:1, :] * inv_n
        var = stats_ref[1:2, :] * inv_n - mean * mean
        scale = gb_ref[0:1, :] * jax.lax.rsqrt(var + EPS)
        shift = gb_ref[1:2, :] - mean * scale
        hb = h_ref[pl.ds(j * BM2, BM2), :]
        t = jnp.tanh(hb * scale + shift)
        out = jnp.dot(self_ref[...], w2at_ref[...],
                      preferred_element_type=jnp.float32)
        out += jnp.dot(t, w2bt_ref[...], preferred_element_type=jnp.float32)
        out_ref[...] = jnp.tanh(out)


def kernel(seq_self, seq, adj, W1, W2, gamma, beta):
    w1t = W1.T
    w2at = W2[:, :F].T
    w2bt = W2[:, F:].T
    gb = jnp.zeros((8, F), jnp.float32).at[0].set(gamma).at[1].set(beta)

    out = pl.pallas_call(
        _fused_kernel,
        grid=(NP1 + NP2,),
        in_specs=[
            pl.BlockSpec((BM, N), lambda i: (jnp.minimum(i, NP1 - 1), 0)),
            pl.BlockSpec((N, F), lambda i: (0, 0)),
            pl.BlockSpec((F, F), lambda i: (0, 0)),
            pl.BlockSpec((BM2, F), lambda i: (jnp.maximum(i - NP1, 0), 0)),
            pl.BlockSpec((F, F), lambda i: (0, 0)),
            pl.BlockSpec((F, F), lambda i: (0, 0)),
            pl.BlockSpec((8, F), lambda i: (0, 0)),
        ],
        out_specs=pl.BlockSpec((BM2, F), lambda i: (jnp.maximum(i - NP1, 0), 0)),
        out_shape=jax.ShapeDtypeStruct((N, F), jnp.float32),
        scratch_shapes=[
            pltpu.VMEM((N, F), jnp.float32),
            pltpu.VMEM((8, F), jnp.float32),
        ],
        compiler_params=pltpu.CompilerParams(
            dimension_semantics=("arbitrary",)),
    )(adj, seq, w1t, seq_self, w2at, w2bt, gb)
    return out
